# Initial kernel scaffold; baseline (speedup 1.0000x reference)
import jax
import jax.numpy as jnp
from jax.experimental import pallas as pl


def kernel(node_features, incidence, W, b, log_sigma, dW):
    raise NotImplementedError("write your pallas kernel here")


# speedup vs baseline: 1.3740x; 1.3740x over previous
import jax
import jax.numpy as jnp
from jax.experimental import pallas as pl
from jax.experimental.pallas import tpu as pltpu

_N = 10000
_M = 512
_D = 128
_STEPS = 5
_DT = 0.2
_BN = 2000
_NB = _N // _BN


def _sde_kernel(inc_ref, x0_ref, w_ref, b_ref, ls_ref, dw_ref, out_ref,
                e2_ref, eacc_ref, inv_e_ref, inv_v_ref):
    s = pl.program_id(0)
    i = pl.program_id(1)
    row = i * _BN
    inc_blk = inc_ref[pl.ds(row, _BN), :]

    @pl.when(s == 0)
    def _prologue():
        out_ref[pl.ds(row, _BN), :] = x0_ref[pl.ds(row, _BN), :]
        inv_v_ref[pl.ds(row, _BN), :] = (
            1.0 / (jnp.sum(inc_blk, axis=1)[:, None] + 1e-6))
        col = jnp.sum(inc_blk, axis=0)[:, None]

        @pl.when(i == 0)
        def _():
            inv_e_ref[...] = col

        @pl.when(i > 0)
        def _():
            inv_e_ref[...] = inv_e_ref[...] + col

        @pl.when(i == _NB - 1)
        def _():
            inv_e_ref[...] = 1.0 / (inv_e_ref[...] + 1e-6)

    @pl.when(s > 0)
    def _update():
        y = out_ref[pl.ds(row, _BN), :]
        agg = jnp.dot(inc_blk, e2_ref[...],
                      preferred_element_type=jnp.float32)
        agg = agg * inv_v_ref[pl.ds(row, _BN), :]
        sigma = jnp.exp(ls_ref[...])
        out_ref[pl.ds(row, _BN), :] = (
            y + jnp.tanh(agg) * _DT + dw_ref[0, 0] * sigma)

    @pl.when(s < _STEPS)
    def _accumulate():
        ynew = out_ref[pl.ds(row, _BN), :]
        part = jax.lax.dot_general(inc_blk, ynew, (((0,), (0,)), ((), ())),
                                   preferred_element_type=jnp.float32)

        @pl.when(i == 0)
        def _():
            eacc_ref[...] = part

        @pl.when(i > 0)
        def _():
            eacc_ref[...] = eacc_ref[...] + part

        @pl.when(i == _NB - 1)
        def _():
            e2_ref[...] = jnp.dot(eacc_ref[...] * inv_e_ref[...], w_ref[...],
                                  preferred_element_type=jnp.float32) + b_ref[...]


def kernel(node_features, incidence, W, b, log_sigma, dW):
    out = pl.pallas_call(
        _sde_kernel,
        grid=(_STEPS + 1, _NB),
        in_specs=[
            pl.BlockSpec((_N, _M), lambda s, i: (0, 0)),
            pl.BlockSpec((_N, _D), lambda s, i: (0, 0)),
            pl.BlockSpec((_D, _D), lambda s, i: (0, 0)),
            pl.BlockSpec((1, _D), lambda s, i: (0, 0)),
            pl.BlockSpec((1, _D), lambda s, i: (0, 0)),
            pl.BlockSpec((1, 1, _BN, _D),
                         lambda s, i: (jnp.maximum(s - 1, 0), i, 0, 0)),
        ],
        out_specs=pl.BlockSpec((_N, _D), lambda s, i: (0, 0)),
        out_shape=jax.ShapeDtypeStruct((_N, _D), jnp.float32),
        scratch_shapes=[
            pltpu.VMEM((_M, _D), jnp.float32),
            pltpu.VMEM((_M, _D), jnp.float32),
            pltpu.VMEM((_M, 1), jnp.float32),
            pltpu.VMEM((_N, 1), jnp.float32),
        ],
        compiler_params=pltpu.CompilerParams(
            dimension_semantics=("arbitrary", "arbitrary"),
            vmem_limit_bytes=100 * 1024 * 1024,
        ),
    )(incidence, node_features, W, b.reshape(1, _D),
      log_sigma.reshape(1, _D), dW.reshape(_STEPS, _NB, _BN, _D))
    return out.reshape(1, _N * _D)


# speedup vs baseline: 1.4788x; 1.0762x over previous
import jax
import jax.numpy as jnp
from jax.experimental import pallas as pl
from jax.experimental.pallas import tpu as pltpu

_N = 10000
_M = 512
_D = 128
_STEPS = 5
_DT = 0.2
_BN = 2000
_NB = _N // _BN


def _sde_kernel(inc_ref, x0_ref, w_ref, b_ref, ls_ref, dw_ref, out_ref,
                e2_ref, eacc_ref, inv_e_ref, inv_v_ref):
    s = pl.program_id(0)
    i = pl.program_id(1)
    row = i * _BN
    inc_blk = inc_ref[pl.ds(row, _BN), :]

    @pl.when(s == 0)
    def _prologue():
        out_ref[pl.ds(row, _BN), :] = x0_ref[pl.ds(row, _BN), :]
        inv_v_ref[pl.ds(row, _BN), :] = (
            1.0 / (jnp.sum(inc_blk, axis=1, dtype=jnp.float32)[:, None] + 1e-6))
        col = jnp.sum(inc_blk, axis=0, dtype=jnp.float32)[:, None]

        @pl.when(i == 0)
        def _():
            inv_e_ref[...] = col

        @pl.when(i > 0)
        def _():
            inv_e_ref[...] = inv_e_ref[...] + col

        @pl.when(i == _NB - 1)
        def _():
            inv_e_ref[...] = 1.0 / (inv_e_ref[...] + 1e-6)

    @pl.when(s > 0)
    def _update():
        y = out_ref[pl.ds(row, _BN), :]
        agg = jnp.dot(inc_blk, e2_ref[...],
                      preferred_element_type=jnp.float32)
        agg = agg * inv_v_ref[pl.ds(row, _BN), :]
        sigma = jnp.exp(ls_ref[...])
        out_ref[pl.ds(row, _BN), :] = (
            y + jnp.tanh(agg) * _DT + dw_ref[0, 0] * sigma)

    @pl.when(s < _STEPS)
    def _accumulate():
        ynew = out_ref[pl.ds(row, _BN), :].astype(jnp.bfloat16)
        part = jax.lax.dot_general(inc_blk, ynew, (((0,), (0,)), ((), ())),
                                   preferred_element_type=jnp.float32)

        @pl.when(i == 0)
        def _():
            eacc_ref[...] = part

        @pl.when(i > 0)
        def _():
            eacc_ref[...] = eacc_ref[...] + part

        @pl.when(i == _NB - 1)
        def _():
            e2 = jnp.dot(eacc_ref[...] * inv_e_ref[...], w_ref[...],
                         preferred_element_type=jnp.float32) + b_ref[...]
            e2_ref[...] = e2.astype(jnp.bfloat16)


def kernel(node_features, incidence, W, b, log_sigma, dW):
    out = pl.pallas_call(
        _sde_kernel,
        grid=(_STEPS + 1, _NB),
        in_specs=[
            pl.BlockSpec((_N, _M), lambda s, i: (0, 0)),
            pl.BlockSpec((_N, _D), lambda s, i: (0, 0)),
            pl.BlockSpec((_D, _D), lambda s, i: (0, 0)),
            pl.BlockSpec((1, _D), lambda s, i: (0, 0)),
            pl.BlockSpec((1, _D), lambda s, i: (0, 0)),
            pl.BlockSpec((1, 1, _BN, _D),
                         lambda s, i: (jnp.maximum(s - 1, 0), i, 0, 0)),
        ],
        out_specs=pl.BlockSpec((_N, _D), lambda s, i: (0, 0)),
        out_shape=jax.ShapeDtypeStruct((_N, _D), jnp.float32),
        scratch_shapes=[
            pltpu.VMEM((_M, _D), jnp.bfloat16),
            pltpu.VMEM((_M, _D), jnp.float32),
            pltpu.VMEM((_M, 1), jnp.float32),
            pltpu.VMEM((_N, 1), jnp.float32),
        ],
        compiler_params=pltpu.CompilerParams(
            dimension_semantics=("arbitrary", "arbitrary"),
            vmem_limit_bytes=100 * 1024 * 1024,
        ),
    )(incidence.astype(jnp.bfloat16), node_features, W, b.reshape(1, _D),
      log_sigma.reshape(1, _D), dW.reshape(_STEPS, _NB, _BN, _D))
    return out.reshape(1, _N * _D)


# speedup vs baseline: 1.5531x; 1.0503x over previous
import jax
import jax.numpy as jnp
from jax.experimental import pallas as pl
from jax.experimental.pallas import tpu as pltpu

_N = 10000
_M = 512
_D = 128
_STEPS = 5
_DT = 0.2
_BN = 2000
_NB = _N // _BN


def _sde_kernel(inc_ref, x0_ref, w_ref, b_ref, ls_ref, dw_ref, out_ref,
                incb_ref, inct_ref, e2_ref, eacc_ref, inv_e_ref, inv_v_ref):
    s = pl.program_id(0)
    i = pl.program_id(1)
    row = i * _BN

    @pl.when(s == 0)
    def _prologue():
        inc_f = inc_ref[...]
        out_ref[pl.ds(row, _BN), :] = x0_ref[...]
        inv_v_ref[pl.ds(row, _BN), :] = (
            1.0 / (jnp.sum(inc_f, axis=1)[:, None] + 1e-6))
        col = jnp.sum(inc_f, axis=0)[:, None]

        @pl.when(i == 0)
        def _():
            inv_e_ref[...] = col

        @pl.when(i > 0)
        def _():
            inv_e_ref[...] = inv_e_ref[...] + col

        @pl.when(i == _NB - 1)
        def _():
            inv_e_ref[...] = 1.0 / (inv_e_ref[...] + 1e-6)

        inc_bf = inc_f.astype(jnp.bfloat16)
        incb_ref[pl.ds(row, _BN), :] = inc_bf
        inct_ref[i] = jnp.transpose(inc_bf)

    @pl.when(s > 0)
    def _update():
        y = out_ref[pl.ds(row, _BN), :]
        agg = jnp.dot(incb_ref[pl.ds(row, _BN), :], e2_ref[...],
                      preferred_element_type=jnp.float32)
        agg = agg * inv_v_ref[pl.ds(row, _BN), :]
        sigma = jnp.exp(ls_ref[...])
        out_ref[pl.ds(row, _BN), :] = (
            y + jnp.tanh(agg) * _DT + dw_ref[0, 0] * sigma)

    @pl.when(s < _STEPS)
    def _accumulate():
        ynew = out_ref[pl.ds(row, _BN), :].astype(jnp.bfloat16)
        part = jnp.dot(inct_ref[i], ynew, preferred_element_type=jnp.float32)

        @pl.when(i == 0)
        def _():
            eacc_ref[...] = part

        @pl.when(i > 0)
        def _():
            eacc_ref[...] = eacc_ref[...] + part

        @pl.when(i == _NB - 1)
        def _():
            e2 = jnp.dot(eacc_ref[...] * inv_e_ref[...], w_ref[...],
                         preferred_element_type=jnp.float32) + b_ref[...]
            e2_ref[...] = e2.astype(jnp.bfloat16)


def kernel(node_features, incidence, W, b, log_sigma, dW):
    out = pl.pallas_call(
        _sde_kernel,
        grid=(_STEPS + 1, _NB),
        in_specs=[
            pl.BlockSpec((_BN, _M),
                         lambda s, i: (jnp.where(s == 0, i, _NB - 1), 0)),
            pl.BlockSpec((_BN, _D),
                         lambda s, i: (jnp.where(s == 0, i, _NB - 1), 0)),
            pl.BlockSpec((_D, _D), lambda s, i: (0, 0)),
            pl.BlockSpec((1, _D), lambda s, i: (0, 0)),
            pl.BlockSpec((1, _D), lambda s, i: (0, 0)),
            pl.BlockSpec((1, 1, _BN, _D),
                         lambda s, i: (jnp.maximum(s - 1, 0), i, 0, 0)),
        ],
        out_specs=pl.BlockSpec((_N, _D), lambda s, i: (0, 0)),
        out_shape=jax.ShapeDtypeStruct((_N, _D), jnp.float32),
        scratch_shapes=[
            pltpu.VMEM((_N, _M), jnp.bfloat16),
            pltpu.VMEM((_NB, _M, _BN), jnp.bfloat16),
            pltpu.VMEM((_M, _D), jnp.bfloat16),
            pltpu.VMEM((_M, _D), jnp.float32),
            pltpu.VMEM((_M, 1), jnp.float32),
            pltpu.VMEM((_N, 1), jnp.float32),
        ],
        compiler_params=pltpu.CompilerParams(
            dimension_semantics=("arbitrary", "arbitrary"),
            vmem_limit_bytes=100 * 1024 * 1024,
        ),
    )(incidence, node_features, W, b.reshape(1, _D),
      log_sigma.reshape(1, _D), dW.reshape(_STEPS, _NB, _BN, _D))
    return out.reshape(1, _N * _D)
